# Initial kernel scaffold; baseline (speedup 1.0000x reference)
import jax
import jax.numpy as jnp
from jax.experimental import pallas as pl


def kernel(x, edge_index, W0, a_s0, a_d0, b0, W1, a_s1, a_d1, b1, Wc, bc):
    raise NotImplementedError("write your pallas kernel here")


# speedup vs baseline: 44.5475x; 44.5475x over previous
import functools

import jax
import jax.numpy as jnp
from jax import lax
from jax.experimental import pallas as pl
from jax.experimental.pallas import tpu as pltpu, tpu_sc as plsc

N = 10000
HEADS = 2
HID = 10
E_REAL = 330000
NW = 32
CH = 128
NCH = 82
PT = CH * NCH
E_PAD = NW * PT
ACC_ROWS = 10240
DUMMY = 10008
B_ROWS = 10016
AW = 32
BW = 16


def _leaky(v):
    return jnp.where(v >= 0.0, v, 0.2 * v)


def _tc0_body(x_ref, wa_ref, wb_ref, a_ref, b_ref, ma_ref, mb_ref):
    xb = x_ref[...]
    a = jnp.dot(xb, wa_ref[...], preferred_element_type=jnp.float32)
    bb = jnp.dot(xb, wb_ref[...], preferred_element_type=jnp.float32)
    a_ref[...] = a
    b_ref[...] = bb
    ma = jnp.max(a, axis=0, keepdims=True)
    mb = jnp.max(bb, axis=0, keepdims=True)

    @pl.when(pl.program_id(0) == 0)
    def _():
        ma_ref[...] = ma
        mb_ref[...] = mb

    @pl.when(pl.program_id(0) != 0)
    def _():
        ma_ref[...] = jnp.maximum(ma_ref[...], ma)
        mb_ref[...] = jnp.maximum(mb_ref[...], mb)


def _tc0(x, waug, wb):
    bn = 1000
    return pl.pallas_call(
        _tc0_body,
        grid=(N // bn,),
        in_specs=[
            pl.BlockSpec((bn, 128), lambda i: (i, 0)),
            pl.BlockSpec((128, AW), lambda i: (0, 0)),
            pl.BlockSpec((128, BW), lambda i: (0, 0)),
        ],
        out_specs=[
            pl.BlockSpec((bn, AW), lambda i: (i, 0)),
            pl.BlockSpec((bn, BW), lambda i: (i, 0)),
            pl.BlockSpec((1, AW), lambda i: (0, 0)),
            pl.BlockSpec((1, BW), lambda i: (0, 0)),
        ],
        out_shape=[
            jax.ShapeDtypeStruct((N, AW), jnp.float32),
            jax.ShapeDtypeStruct((N, BW), jnp.float32),
            jax.ShapeDtypeStruct((1, AW), jnp.float32),
            jax.ShapeDtypeStruct((1, BW), jnp.float32),
        ],
    )(x, waug, wb)


def _sc_body(src_h, dst_h, a_h, b_h, cvec_h, zeros_h, out_h,
             acc_sh, src_v, dst_v, a_v, b_v, c_v, cv_v, z_v, sem1, sem2):
    c = lax.axis_index("c")
    s = lax.axis_index("s")
    w = s * 2 + c
    pltpu.sync_copy(cvec_h, cv_v)
    pltpu.sync_copy(zeros_h, z_v)
    for j in range(5):
        pltpu.sync_copy(z_v, acc_sh.at[pl.ds(s * 640 + j * CH, CH)])
    plsc.subcore_barrier()

    c0 = cv_v[pl.ds(0, 16)]
    c1 = cv_v[pl.ds(16, 16)]
    iota = jnp.arange(16, dtype=jnp.int32)
    ia = iota * AW
    ib = iota * BW
    base_e = w * PT

    def chunk(i, carry):
        off = base_e + i * CH
        pltpu.sync_copy(src_h.at[pl.ds(off, CH)], src_v)
        pltpu.sync_copy(dst_h.at[pl.ds(off, CH)], dst_v)
        pltpu.async_copy(a_h.at[src_v], a_v, sem1).wait()
        pltpu.async_copy(b_h.at[dst_v], b_v, sem2).wait()
        for g in range(CH // 16):
            row = g * 16 + iota
            f = lambda v: jnp.full((16,), v, jnp.int32)
            as0 = plsc.load_gather(a_v, [row, f(20)])
            as1 = plsc.load_gather(a_v, [row, f(21)])
            ad0 = plsc.load_gather(b_v, [row, f(0)])
            ad1 = plsc.load_gather(b_v, [row, f(1)])
            w0 = jnp.exp(_leaky(as0 + ad0) - c0)
            w1 = jnp.exp(_leaky(as1 + ad1) - c1)
            for cc in range(2 * HID):
                av = plsc.load_gather(a_v, [row, f(cc)])
                plsc.store_scatter(c_v, [row, f(cc)],
                                   av * (w0 if cc < HID else w1))
            plsc.store_scatter(c_v, [row, f(20)], w0)
            plsc.store_scatter(c_v, [row, f(21)], w1)
        pltpu.sync_copy(c_v, acc_sh.at[dst_v], add=True)
        return carry

    lax.fori_loop(0, NCH, chunk, 0)
    plsc.subcore_barrier()
    pltpu.sync_copy(acc_sh.at[pl.ds(s * 640, 640)],
                    out_h.at[c, pl.ds(s * 640, 640)])


_sc_edge = functools.partial(
    pl.kernel,
    mesh=plsc.VectorSubcoreMesh(core_axis_name="c", subcore_axis_name="s"),
    out_type=jax.ShapeDtypeStruct((2, ACC_ROWS, AW), jnp.float32),
    scratch_types=[
        pltpu.VMEM_SHARED((ACC_ROWS, AW), jnp.float32),
        pltpu.VMEM((CH,), jnp.int32),
        pltpu.VMEM((CH,), jnp.int32),
        pltpu.VMEM((CH, AW), jnp.float32),
        pltpu.VMEM((CH, BW), jnp.float32),
        pltpu.VMEM((CH, AW), jnp.float32),
        pltpu.VMEM((32,), jnp.float32),
        pltpu.VMEM((CH, AW), jnp.float32),
        pltpu.SemaphoreType.DMA,
        pltpu.SemaphoreType.DMA,
    ],
    compiler_params=pltpu.CompilerParams(needs_layout_passes=False,
                                         use_tc_tiling_on_sc=False),
)(_sc_body)


def _tc1_body(p_ref, b0_ref, wa_ref, wb_ref, a_ref, bt_ref, ma_ref, mb_ref):
    acc = p_ref[0] + p_ref[1]
    col = lax.broadcasted_iota(jnp.int32, acc.shape, 1)
    den = jnp.where(col < HID, acc[:, 20:21], acc[:, 21:22])
    g = acc / (den + 1e-30) + b0_ref[...]
    g = jnp.where(g > 0.0, g, jnp.exp(g) - 1.0)
    g = jnp.where(col < 2 * HID, g, 0.0)
    a = jnp.dot(g, wa_ref[...], preferred_element_type=jnp.float32)
    bb = jnp.dot(g, wb_ref[...], preferred_element_type=jnp.float32)
    a_ref[...] = a
    bt_ref[...] = bb
    ma = jnp.max(a, axis=0, keepdims=True)
    mb = jnp.max(bb, axis=0, keepdims=True)

    @pl.when(pl.program_id(0) == 0)
    def _():
        ma_ref[...] = ma
        mb_ref[...] = mb

    @pl.when(pl.program_id(0) != 0)
    def _():
        ma_ref[...] = jnp.maximum(ma_ref[...], ma)
        mb_ref[...] = jnp.maximum(mb_ref[...], mb)


def _tc1(p, b0p, w1aug, w1b):
    bn = 1000
    return pl.pallas_call(
        _tc1_body,
        grid=(N // bn,),
        in_specs=[
            pl.BlockSpec((2, bn, AW), lambda i: (0, i, 0)),
            pl.BlockSpec((1, AW), lambda i: (0, 0)),
            pl.BlockSpec((AW, AW), lambda i: (0, 0)),
            pl.BlockSpec((AW, BW), lambda i: (0, 0)),
        ],
        out_specs=[
            pl.BlockSpec((bn, AW), lambda i: (i, 0)),
            pl.BlockSpec((bn, BW), lambda i: (i, 0)),
            pl.BlockSpec((1, AW), lambda i: (0, 0)),
            pl.BlockSpec((1, BW), lambda i: (0, 0)),
        ],
        out_shape=[
            jax.ShapeDtypeStruct((N, AW), jnp.float32),
            jax.ShapeDtypeStruct((N, BW), jnp.float32),
            jax.ShapeDtypeStruct((1, AW), jnp.float32),
            jax.ShapeDtypeStruct((1, BW), jnp.float32),
        ],
    )(p, b0p, w1aug, w1b)


def _tc2_body(p_ref, b1_ref, wc_ref, bc_ref, o_ref, sum_ref):
    i = pl.program_id(0)
    acc = p_ref[0] + p_ref[1]
    col = lax.broadcasted_iota(jnp.int32, acc.shape, 1)
    den = jnp.where(col < HID, acc[:, 20:21], acc[:, 21:22])
    v = acc / (den + 1e-30)
    v = jnp.where(col < 2 * HID, v, 0.0)
    s = jnp.sum(v, axis=0, keepdims=True)

    @pl.when(i == 0)
    def _():
        sum_ref[...] = s

    @pl.when(i != 0)
    def _():
        sum_ref[...] = sum_ref[...] + s

    @pl.when(i == pl.num_programs(0) - 1)
    def _():
        g = sum_ref[...] / float(N) + b1_ref[...]
        logits = jnp.dot(g, wc_ref[...], preferred_element_type=jnp.float32)
        logits = logits + bc_ref[...]
        m = jnp.max(logits, axis=1, keepdims=True)
        e = jnp.exp(logits - m)
        o_ref[...] = e / jnp.sum(e, axis=1, keepdims=True)


def _tc2(p, b1p, wcp, bcp):
    bn = 1000
    return pl.pallas_call(
        _tc2_body,
        grid=(N // bn,),
        in_specs=[
            pl.BlockSpec((2, bn, AW), lambda i: (0, i, 0)),
            pl.BlockSpec((1, AW), lambda i: (0, 0)),
            pl.BlockSpec((AW, AW), lambda i: (0, 0)),
            pl.BlockSpec((1, AW), lambda i: (0, 0)),
        ],
        out_specs=pl.BlockSpec((1, AW), lambda i: (0, 0)),
        out_shape=jax.ShapeDtypeStruct((1, AW), jnp.float32),
        scratch_shapes=[pltpu.VMEM((1, AW), jnp.float32)],
    )(p, b1p, wcp, bcp)


def _aug_weights(W, a_s, a_d):
    fin = W.shape[0]
    wa = jnp.zeros((fin, AW), jnp.float32)
    wa = wa.at[:, 0:2 * HID].set(W)
    wa = wa.at[:, 20].set(W[:, 0:HID] @ a_s[0])
    wa = wa.at[:, 21].set(W[:, HID:2 * HID] @ a_s[1])
    wb = jnp.zeros((fin, BW), jnp.float32)
    wb = wb.at[:, 0].set(W[:, 0:HID] @ a_d[0])
    wb = wb.at[:, 1].set(W[:, HID:2 * HID] @ a_d[1])
    return wa, wb


def _cvec(ma, mb):
    c0 = _leaky(ma[0, 20] + mb[0, 0])
    c1 = _leaky(ma[0, 21] + mb[0, 1])
    return jnp.concatenate([jnp.full((16,), c0, jnp.float32),
                            jnp.full((16,), c1, jnp.float32)])


def _pad_vec(v):
    return jnp.zeros((1, AW), jnp.float32).at[0, 0:v.shape[0]].set(v)


def kernel(x, edge_index, W0, a_s0, a_d0, b0, W1, a_s1, a_d1, b1, Wc, bc):
    loops = jnp.arange(N, dtype=jnp.int32)
    src = jnp.concatenate([edge_index[0].astype(jnp.int32), loops,
                           jnp.zeros((E_PAD - E_REAL,), jnp.int32)])
    dst = jnp.concatenate([edge_index[1].astype(jnp.int32), loops,
                           jnp.full((E_PAD - E_REAL,), DUMMY, jnp.int32)])
    zeros_blk = jnp.zeros((CH, AW), jnp.float32)

    wa0, wb0 = _aug_weights(W0, a_s0, a_d0)
    a_t, b_t, ma, mb = _tc0(x, wa0, wb0)
    b_t = jnp.zeros((B_ROWS, BW), jnp.float32).at[0:N].set(b_t)
    p0 = _sc_edge(src, dst, a_t, b_t, _cvec(ma, mb), zeros_blk)

    wa1, wb1 = _aug_weights(W1, a_s1, a_d1)
    wa1 = jnp.zeros((AW, AW), jnp.float32).at[0:2 * HID].set(wa1)
    wb1 = jnp.zeros((AW, BW), jnp.float32).at[0:2 * HID].set(wb1)
    a_t1, b_t1, ma1, mb1 = _tc1(p0, _pad_vec(b0), wa1, wb1)
    b_t1 = jnp.zeros((B_ROWS, BW), jnp.float32).at[0:N].set(b_t1)
    p1 = _sc_edge(src, dst, a_t1, b_t1, _cvec(ma1, mb1), zeros_blk)

    wcp = jnp.zeros((AW, AW), jnp.float32).at[0:2 * HID].set(Wc)
    out = _tc2(p1, _pad_vec(b1), wcp, _pad_vec(bc))
    return out[:, 0:32]
